# baseline (speedup 1.0000x reference)
import jax
import jax.numpy as jnp
from jax import lax
from jax.experimental import pallas as pl
from jax.experimental.pallas import tpu as pltpu
from jax.experimental.pallas import tpu_sc as plsc

_NC = 2
_NS = 16
_NW = _NC * _NS

_B, _N, _S, _D = 4, 16, 2048, 128
_BSC = 2
_ROW = _S * _D
_PIECE = 65536
_ROWP = _ROW // _PIECE
_NJ = 60

_NSTEP = _ROWP + 1


def _idx_of(j):
    h, m = j // 30, j % 30
    head = jnp.where(h == 0, m, 3 * (m % 3) + m // 3)
    t, g = (m - 9) % 3, (m - 9) // 3
    pair = jnp.where(h == 0, 6 + t, 2 + 3 * t)
    tail = jnp.where(t < 2, pair, g + 9)
    return jnp.where(m < 9, head, tail)


def _step_offsets(w, k):
    if k < _ROWP:
        p = k
        b, i = w // _N, w % _N
        src = ((b * _N + i) * _ROWP + p) * _PIECE
        j1 = jnp.where(i < 9, i, 3 * i - 16)
        j2 = jnp.where(i < 9, 30 + 3 * (i % 3) + i // 3, 3 * i + 14)
        dsts = [((b * _NJ + j) * _ROWP + p) * _PIECE for j in (j1, j2)]
    else:
        h, p = w // 4, w % 4
        b, e = h // 4, h % 4
        src_row = jnp.where(e < 2, 6 + e, 3 * e - 4)
        j0 = jnp.where(e < 2, 9 + e, 37 + e)
        src = ((b * _N + src_row) * _ROWP + p) * _PIECE
        dsts = [((b * _NJ + j0 + 3 * t) * _ROWP + p) * _PIECE
                for t in range(7)]
    return src, dsts


def _sc_body(x_hbm, out_hbm, buf0, buf1, sg0, sg1, ss0, ss1):
    c = lax.axis_index("c")
    s = lax.axis_index("s")
    w = s * _NC + c
    bufs = (buf0.at[s], buf1)
    sgs, sss = (sg0, sg1), (ss0, ss1)

    def n_stores(k):
        return 2 if k < _ROWP else 7

    def start_gather(k, b):
        src, _ = _step_offsets(w, k)
        pltpu.async_copy(x_hbm.at[pl.ds(pl.multiple_of(src, _PIECE), _PIECE)],
                         bufs[b], sgs[b])

    def wait_gather(b):
        pltpu.make_async_copy(x_hbm.at[pl.ds(0, _PIECE)], bufs[b],
                              sgs[b]).wait()

    def start_stores(k, b):
        _, dsts = _step_offsets(w, k)
        for d in dsts:
            pltpu.async_copy(
                bufs[b], out_hbm.at[pl.ds(pl.multiple_of(d, _PIECE), _PIECE)],
                sss[b])

    def wait_stores(k, b):
        for _ in range(n_stores(k)):
            pltpu.make_async_copy(bufs[b], out_hbm.at[pl.ds(0, _PIECE)],
                                  sss[b]).wait()

    order = [_ROWP] + list(range(_ROWP))
    start_gather(order[0], 0)
    for p, k in enumerate(order):
        b = p % 2
        wait_gather(b)
        start_stores(k, b)
        if p + 1 < _NSTEP:
            if p >= 1:
                wait_stores(order[p - 1], 1 - b)
            start_gather(order[p + 1], 1 - b)
    wait_stores(order[_NSTEP - 2], _NSTEP % 2)
    wait_stores(order[_NSTEP - 1], (_NSTEP - 1) % 2)


def _sc_part(x1):
    mesh = plsc.VectorSubcoreMesh(core_axis_name="c", subcore_axis_name="s")
    return pl.kernel(
        _sc_body,
        out_type=jax.ShapeDtypeStruct((_BSC * _NJ * _ROW,), jnp.float32),
        mesh=mesh,
        scratch_types=[
            pltpu.VMEM_SHARED((_NS, _PIECE), jnp.float32),
            pltpu.VMEM((_PIECE,), jnp.float32),
            pltpu.SemaphoreType.DMA,
            pltpu.SemaphoreType.DMA,
            pltpu.SemaphoreType.DMA,
            pltpu.SemaphoreType.DMA,
        ],
    )(x1)


def _tc_body(x_ref, o_ref):
    o_ref[...] = x_ref[...]


def _tc_part(x3):
    nrows = (_B - _BSC) * _NJ

    def in_map(jj):
        b = _BSC + jj // _NJ
        return b * _N + _idx_of(jj % _NJ), 0, 0

    return pl.pallas_call(
        _tc_body,
        grid=(nrows,),
        in_specs=[pl.BlockSpec((1, _S, _D), in_map)],
        out_specs=pl.BlockSpec((1, _S, _D), lambda jj: (jj, 0, 0)),
        out_shape=jax.ShapeDtypeStruct((nrows, _S, _D), jnp.float32),
    )(x3)


@jax.jit
def kernel(x):
    b, n, s, d = x.shape
    sc = _sc_part(x.reshape(-1))
    tc = _tc_part(x.reshape(b * n, s, d))
    out = jnp.concatenate([sc.reshape(_BSC * _NJ, s, d), tc], axis=0)
    return out.reshape(b, _NJ // 3, 3, s, d)


# speedup vs baseline: 2.4255x; 2.4255x over previous
import jax
import jax.numpy as jnp
from jax import lax
from jax.experimental import pallas as pl
from jax.experimental.pallas import tpu as pltpu
from jax.experimental.pallas import tpu_sc as plsc

_NC = 2
_NS = 16
_NW = _NC * _NS

_B, _N, _S, _D = 4, 16, 2048, 128
_ROW = _S * _D
_PIECE = 65536
_ROWP = _ROW // _PIECE
_NJ = 60
_NQ = _B * _NJ * _ROWP

_NSTEP = 2 * _ROWP + 2


def _step_offsets(w, k):
    if k < 2 * _ROWP:
        u, p = k // _ROWP, k % _ROWP
        n = 2 * w + u
        b, i = n // _N, n % _N
        src = ((b * _N + i) * _ROWP + p) * _PIECE
        j1 = jnp.where(i < 9, i, 3 * i - 16)
        j2 = jnp.where(i < 9, 30 + 3 * (i % 3) + i // 3, 3 * i + 14)
        dsts = [((b * _NJ + j) * _ROWP + p) * _PIECE for j in (j1, j2)]
    else:
        h, half = w // 2, w % 2
        p = 2 * half + (k - 2 * _ROWP)
        b, e = h // 4, h % 4
        src_row = jnp.where(e < 2, 6 + e, 3 * e - 4)
        j0 = jnp.where(e < 2, 9 + e, 37 + e)
        src = ((b * _N + src_row) * _ROWP + p) * _PIECE
        dsts = [((b * _NJ + j0 + 3 * t) * _ROWP + p) * _PIECE
                for t in range(7)]
    return src, dsts


def _body(x_hbm, out_hbm, buf0, buf1, sg0, sg1, ss0, ss1):
    c = lax.axis_index("c")
    s = lax.axis_index("s")
    w = s * _NC + c
    bufs = (buf0.at[s], buf1)
    sgs, sss = (sg0, sg1), (ss0, ss1)

    def n_stores(k):
        return 2 if k < 2 * _ROWP else 7

    def start_gather(k):
        src, _ = _step_offsets(w, k)
        b = k % 2
        pltpu.async_copy(x_hbm.at[pl.ds(pl.multiple_of(src, _PIECE), _PIECE)],
                         bufs[b], sgs[b])

    def wait_gather(k):
        b = k % 2
        pltpu.make_async_copy(x_hbm.at[pl.ds(0, _PIECE)], bufs[b],
                              sgs[b]).wait()

    def start_stores(k):
        _, dsts = _step_offsets(w, k)
        b = k % 2
        for d in dsts:
            pltpu.async_copy(
                bufs[b], out_hbm.at[pl.ds(pl.multiple_of(d, _PIECE), _PIECE)],
                sss[b])

    def wait_stores(k):
        b = k % 2
        for _ in range(n_stores(k)):
            pltpu.make_async_copy(bufs[b], out_hbm.at[pl.ds(0, _PIECE)],
                                  sss[b]).wait()

    order = [2 * _ROWP, 2 * _ROWP + 1] + list(range(2 * _ROWP))
    start_gather(order[0])
    for p, k in enumerate(order):
        wait_gather(k)
        start_stores(k)
        if p + 1 < _NSTEP:
            if p >= 1:
                wait_stores(order[p - 1])
            start_gather(order[p + 1])
    wait_stores(order[_NSTEP - 2])
    wait_stores(order[_NSTEP - 1])


@jax.jit
def kernel(x):
    b, n, s, d = x.shape
    x1 = x.reshape(-1)
    mesh = plsc.VectorSubcoreMesh(core_axis_name="c", subcore_axis_name="s")
    out = pl.kernel(
        _body,
        out_type=jax.ShapeDtypeStruct((_NQ * _PIECE,), jnp.float32),
        mesh=mesh,
        scratch_types=[
            pltpu.VMEM_SHARED((_NS, _PIECE), jnp.float32),
            pltpu.VMEM((_PIECE,), jnp.float32),
            pltpu.SemaphoreType.DMA,
            pltpu.SemaphoreType.DMA,
            pltpu.SemaphoreType.DMA,
            pltpu.SemaphoreType.DMA,
        ],
    )(x1)
    return out.reshape(b, _NJ // 3, 3, s, d)
